# baseline (speedup 1.0000x reference)
import functools

import jax
import jax.numpy as jnp
from jax.experimental import pallas as pl

N_TOK, D_MODEL, N_NODES, N_EXPERTS = 8192, 1024, 16, 64
LANES = 128
ROWS = 128
M_SUB = LANES // N_NODES
BLOCK_N = 128


def _round_to_bf16_in_f32(x):
    u = jax.lax.bitcast_convert_type(x, jnp.int32)
    rounded = (u + 0x7FFF + ((u >> 16) & 1)) & jnp.int32(-65536)
    return jax.lax.bitcast_convert_type(rounded, jnp.float32)


def _router_body(x_ref, wst_ref, bst_ref, w3_ref, bg_ref, gates_ref, load_ref):
    xr = _round_to_bf16_in_f32(x_ref[...])
    p = xr * wst_ref[...]
    pt = jnp.swapaxes(p, 1, 2)
    s = jnp.sum(pt.reshape(BLOCK_N, M_SUB, N_NODES, LANES), axis=2)
    s = s + bst_ref[0, 0]
    lm = jax.lax.dot_general(
        s.astype(jnp.bfloat16), w3_ref[...],
        (((2,), (1,)), ((1,), (0,))),
        preferred_element_type=jnp.float32)
    logits = jnp.sum(lm, axis=0) + bg_ref[...]
    col = jax.lax.broadcasted_iota(jnp.int32, logits.shape, 1)
    m1 = jnp.max(logits, axis=1, keepdims=True)
    i1 = jnp.min(jnp.where(logits == m1, col, N_EXPERTS), axis=1, keepdims=True)
    masked = jnp.where(col == i1, -jnp.inf, logits)
    m2 = jnp.max(masked, axis=1, keepdims=True)
    i2 = jnp.min(jnp.where(masked == m2, col, N_EXPERTS), axis=1, keepdims=True)
    t = jnp.exp(m2 - m1)
    denom = 1.0 + t
    g1 = 1.0 / denom
    g2 = t / denom
    gates = jnp.where(col == i1, g1, 0.0) + jnp.where(col == i2, g2, 0.0)
    gates_ref[...] = gates
    part = jnp.sum((gates > 0.0).astype(jnp.int32), axis=0, keepdims=True)

    @pl.when(pl.program_id(0) == 0)
    def _init():
        load_ref[...] = part

    @pl.when(pl.program_id(0) != 0)
    def _acc():
        load_ref[...] += part


@functools.partial(jax.jit, static_argnames=("interpret",))
def _run(x3, wst, bst, w3, bg, interpret=False):
    grid = (N_TOK // BLOCK_N,)
    gates, load = pl.pallas_call(
        _router_body,
        grid=grid,
        in_specs=[
            pl.BlockSpec((BLOCK_N, ROWS, LANES), lambda i: (i, 0, 0)),
            pl.BlockSpec((1, 1, LANES), lambda i: (0, 0, 0)),
            pl.BlockSpec((1, 1), lambda i: (0, 0)),
            pl.BlockSpec((M_SUB, LANES, N_EXPERTS), lambda i: (0, 0, 0)),
            pl.BlockSpec((1, N_EXPERTS), lambda i: (0, 0)),
        ],
        out_specs=[
            pl.BlockSpec((BLOCK_N, N_EXPERTS), lambda i: (i, 0)),
            pl.BlockSpec((1, N_EXPERTS), lambda i: (0, 0)),
        ],
        out_shape=[
            jax.ShapeDtypeStruct((N_TOK, N_EXPERTS), jnp.float32),
            jax.ShapeDtypeStruct((1, N_EXPERTS), jnp.int32),
        ],
        interpret=interpret,
    )(x3, wst, bst, w3, bg)
    return gates, load[0]


def _prep(x_trans, W_start, b_start, W_gate, b_gate):
    x3 = x_trans.reshape(N_TOK, ROWS, LANES)
    wsb = jax.lax.reduce_precision(W_start[0], 8, 7)
    wst = jnp.tile(wsb, M_SUB).reshape(1, 1, LANES)
    bst = jnp.reshape(b_start[0], (1, 1)).astype(jnp.float32)
    w3 = (W_gate.astype(jnp.bfloat16).T
          .reshape(LANES, M_SUB, N_EXPERTS)
          .transpose(1, 0, 2))
    bg = b_gate.astype(jnp.float32)[None, :]
    return x3, wst, bst, w3, bg


def kernel(x_trans, W_start, b_start, W_gate, b_gate, W_noise, b_noise, train):
    return _run(*_prep(x_trans, W_start, b_start, W_gate, b_gate))


# speedup vs baseline: 4.2613x; 4.2613x over previous
import functools

import jax
import jax.numpy as jnp
from jax.experimental import pallas as pl

N_TOK, D_MODEL, N_NODES, N_EXPERTS = 8192, 1024, 16, 64
BLOCK_N = 128


def _round_to_bf16_in_f32(x):
    u = jax.lax.bitcast_convert_type(x, jnp.int32)
    rounded = (u + 0x7FFF + ((u >> 16) & 1)) & jnp.int32(-65536)
    return jax.lax.bitcast_convert_type(rounded, jnp.float32)


def _router_body(x_ref, wsv_ref, bst_ref, wgt_ref, bg_ref, gates_ref, load_ref):
    xr = _round_to_bf16_in_f32(x_ref[...])
    p = xr * wsv_ref[...]
    x1 = jnp.sum(p, axis=1) + bst_ref[0, 0]
    logits = (
        jnp.dot(x1.astype(jnp.bfloat16), wgt_ref[...],
                preferred_element_type=jnp.float32)
        + bg_ref[...]
    )
    col = jax.lax.broadcasted_iota(jnp.int32, logits.shape, 1)
    m1 = jnp.max(logits, axis=1, keepdims=True)
    i1 = jnp.min(jnp.where(logits == m1, col, N_EXPERTS), axis=1, keepdims=True)
    masked = jnp.where(col == i1, -jnp.inf, logits)
    m2 = jnp.max(masked, axis=1, keepdims=True)
    i2 = jnp.min(jnp.where(masked == m2, col, N_EXPERTS), axis=1, keepdims=True)
    t = jnp.exp(m2 - m1)
    denom = 1.0 + t
    g1 = 1.0 / denom
    g2 = t / denom
    gates = jnp.where(col == i1, g1, 0.0) + jnp.where(col == i2, g2, 0.0)
    gates_ref[...] = gates
    part = jnp.sum((gates > 0.0).astype(jnp.int32), axis=0, keepdims=True)

    @pl.when(pl.program_id(0) == 0)
    def _init():
        load_ref[...] = part

    @pl.when(pl.program_id(0) != 0)
    def _acc():
        load_ref[...] += part


@functools.partial(jax.jit, static_argnames=("interpret",))
def _run(xv, wsv, bst, wgt, bg, interpret=False):
    grid = (N_TOK // BLOCK_N,)
    gates, load = pl.pallas_call(
        _router_body,
        grid=grid,
        in_specs=[
            pl.BlockSpec((BLOCK_N, N_NODES, D_MODEL), lambda i: (i, 0, 0)),
            pl.BlockSpec((1, N_NODES, 1), lambda i: (0, 0, 0)),
            pl.BlockSpec((1, 1), lambda i: (0, 0)),
            pl.BlockSpec((D_MODEL, N_EXPERTS), lambda i: (0, 0)),
            pl.BlockSpec((1, N_EXPERTS), lambda i: (0, 0)),
        ],
        out_specs=[
            pl.BlockSpec((BLOCK_N, N_EXPERTS), lambda i: (i, 0)),
            pl.BlockSpec((1, N_EXPERTS), lambda i: (0, 0)),
        ],
        out_shape=[
            jax.ShapeDtypeStruct((N_TOK, N_EXPERTS), jnp.float32),
            jax.ShapeDtypeStruct((1, N_EXPERTS), jnp.int32),
        ],
        interpret=interpret,
    )(xv, wsv, bst, wgt, bg)
    return gates, load[0]


def _prep(x_trans, W_start, b_start, W_gate, b_gate):
    xv = jnp.transpose(x_trans, (0, 2, 1))
    wsv = jax.lax.reduce_precision(W_start[0], 8, 7).reshape(1, N_NODES, 1)
    bst = jnp.reshape(b_start[0], (1, 1)).astype(jnp.float32)
    wgt = W_gate.T.astype(jnp.bfloat16)
    bg = b_gate.astype(jnp.float32)[None, :]
    return xv, wsv, bst, wgt, bg


def kernel(x_trans, W_start, b_start, W_gate, b_gate, W_noise, b_noise, train):
    return _run(*_prep(x_trans, W_start, b_start, W_gate, b_gate))


# speedup vs baseline: 5.2008x; 1.2205x over previous
import functools

import jax
import jax.numpy as jnp
from jax.experimental import pallas as pl

N_TOK, D_MODEL, N_NODES, N_EXPERTS = 8192, 1024, 16, 64
BLOCK_N = 128


def _round_to_bf16_in_f32(x):
    u = jax.lax.bitcast_convert_type(x, jnp.int32)
    rounded = (u + 0x7FFF + ((u >> 16) & 1)) & jnp.int32(-65536)
    return jax.lax.bitcast_convert_type(rounded, jnp.float32)


def _router_body(x_ref, wsv_ref, bst_ref, wgt_ref, bg_ref, gates_ref, load_ref):
    xr = x_ref[...].astype(jnp.bfloat16).astype(jnp.float32)
    p = xr * wsv_ref[...]
    x1 = jnp.sum(p, axis=1) + bst_ref[0, 0]
    logits = (
        jnp.dot(x1.astype(jnp.bfloat16), wgt_ref[...],
                preferred_element_type=jnp.float32)
        + bg_ref[...]
    )
    col = jax.lax.broadcasted_iota(jnp.int32, logits.shape, 1)
    m1 = jnp.max(logits, axis=1, keepdims=True)
    i1 = jnp.min(jnp.where(logits == m1, col, N_EXPERTS), axis=1, keepdims=True)
    masked = jnp.where(col == i1, -jnp.inf, logits)
    m2 = jnp.max(masked, axis=1, keepdims=True)
    i2 = jnp.min(jnp.where(masked == m2, col, N_EXPERTS), axis=1, keepdims=True)
    t = jnp.exp(m2 - m1)
    denom = 1.0 + t
    g1 = 1.0 / denom
    g2 = t / denom
    gates = jnp.where(col == i1, g1, 0.0) + jnp.where(col == i2, g2, 0.0)
    gates_ref[...] = gates
    part = jnp.sum((gates > 0.0).astype(jnp.int32), axis=0, keepdims=True)

    @pl.when(pl.program_id(0) == 0)
    def _init():
        load_ref[...] = part

    @pl.when(pl.program_id(0) != 0)
    def _acc():
        load_ref[...] += part


@functools.partial(jax.jit, static_argnames=("interpret",))
def _run(xv, wsv, bst, wgt, bg, interpret=False):
    grid = (N_TOK // BLOCK_N,)
    gates, load = pl.pallas_call(
        _router_body,
        grid=grid,
        in_specs=[
            pl.BlockSpec((BLOCK_N, N_NODES, D_MODEL), lambda i: (i, 0, 0)),
            pl.BlockSpec((1, N_NODES, 1), lambda i: (0, 0, 0)),
            pl.BlockSpec((1, 1), lambda i: (0, 0)),
            pl.BlockSpec((D_MODEL, N_EXPERTS), lambda i: (0, 0)),
            pl.BlockSpec((1, N_EXPERTS), lambda i: (0, 0)),
        ],
        out_specs=[
            pl.BlockSpec((BLOCK_N, N_EXPERTS), lambda i: (i, 0)),
            pl.BlockSpec((1, N_EXPERTS), lambda i: (0, 0)),
        ],
        out_shape=[
            jax.ShapeDtypeStruct((N_TOK, N_EXPERTS), jnp.float32),
            jax.ShapeDtypeStruct((1, N_EXPERTS), jnp.int32),
        ],
        interpret=interpret,
    )(xv, wsv, bst, wgt, bg)
    return gates, load[0]


def _prep(x_trans, W_start, b_start, W_gate, b_gate):
    xv = jnp.transpose(x_trans, (0, 2, 1))
    wsv = jax.lax.reduce_precision(W_start[0], 8, 7).reshape(1, N_NODES, 1)
    bst = jnp.reshape(b_start[0], (1, 1)).astype(jnp.float32)
    wgt = W_gate.T.astype(jnp.bfloat16)
    bg = b_gate.astype(jnp.float32)[None, :]
    return xv, wsv, bst, wgt, bg


def kernel(x_trans, W_start, b_start, W_gate, b_gate, W_noise, b_noise, train):
    return _run(*_prep(x_trans, W_start, b_start, W_gate, b_gate))


# speedup vs baseline: 5.8934x; 1.1332x over previous
import functools

import jax
import jax.numpy as jnp
from jax.experimental import pallas as pl

N_TOK, D_MODEL, N_NODES, N_EXPERTS = 8192, 1024, 16, 64
BLOCK_N = 256


def _round_to_bf16_in_f32(x):
    u = jax.lax.bitcast_convert_type(x, jnp.int32)
    rounded = (u + 0x7FFF + ((u >> 16) & 1)) & jnp.int32(-65536)
    return jax.lax.bitcast_convert_type(rounded, jnp.float32)


def _router_body(x_ref, wsv_ref, bst_ref, wgt_ref, bg_ref, gates_ref, load_ref):
    xr = x_ref[...].astype(jnp.bfloat16).astype(jnp.float32)
    p = xr * wsv_ref[...]
    x1 = jnp.sum(p, axis=1) + bst_ref[0, 0]
    logits = (
        jnp.dot(x1.astype(jnp.bfloat16), wgt_ref[...],
                preferred_element_type=jnp.float32)
        + bg_ref[...]
    )
    col = jax.lax.broadcasted_iota(jnp.int32, logits.shape, 1)
    m1 = jnp.max(logits, axis=1, keepdims=True)
    i1 = jnp.min(jnp.where(logits == m1, col, N_EXPERTS), axis=1, keepdims=True)
    masked = jnp.where(col == i1, -jnp.inf, logits)
    m2 = jnp.max(masked, axis=1, keepdims=True)
    i2 = jnp.min(jnp.where(masked == m2, col, N_EXPERTS), axis=1, keepdims=True)
    t = jnp.exp(m2 - m1)
    denom = 1.0 + t
    g1 = 1.0 / denom
    g2 = t / denom
    gates = jnp.where(col == i1, g1, 0.0) + jnp.where(col == i2, g2, 0.0)
    gates_ref[...] = gates
    part = jnp.sum((gates > 0.0).astype(jnp.int32), axis=0, keepdims=True)

    @pl.when(pl.program_id(0) == 0)
    def _init():
        load_ref[...] = part

    @pl.when(pl.program_id(0) != 0)
    def _acc():
        load_ref[...] += part


@functools.partial(jax.jit, static_argnames=("interpret",))
def _run(xv, wsv, bst, wgt, bg, interpret=False):
    grid = (N_TOK // BLOCK_N,)
    gates, load = pl.pallas_call(
        _router_body,
        grid=grid,
        in_specs=[
            pl.BlockSpec((BLOCK_N, N_NODES, D_MODEL), lambda i: (i, 0, 0)),
            pl.BlockSpec((1, N_NODES, 1), lambda i: (0, 0, 0)),
            pl.BlockSpec((1, 1), lambda i: (0, 0)),
            pl.BlockSpec((D_MODEL, N_EXPERTS), lambda i: (0, 0)),
            pl.BlockSpec((1, N_EXPERTS), lambda i: (0, 0)),
        ],
        out_specs=[
            pl.BlockSpec((BLOCK_N, N_EXPERTS), lambda i: (i, 0)),
            pl.BlockSpec((1, N_EXPERTS), lambda i: (0, 0)),
        ],
        out_shape=[
            jax.ShapeDtypeStruct((N_TOK, N_EXPERTS), jnp.float32),
            jax.ShapeDtypeStruct((1, N_EXPERTS), jnp.int32),
        ],
        interpret=interpret,
    )(xv, wsv, bst, wgt, bg)
    return gates, load[0]


def _prep(x_trans, W_start, b_start, W_gate, b_gate):
    xv = jnp.transpose(x_trans, (0, 2, 1))
    wsv = jax.lax.reduce_precision(W_start[0], 8, 7).reshape(1, N_NODES, 1)
    bst = jnp.reshape(b_start[0], (1, 1)).astype(jnp.float32)
    wgt = W_gate.T.astype(jnp.bfloat16)
    bg = b_gate.astype(jnp.float32)[None, :]
    return xv, wsv, bst, wgt, bg


def kernel(x_trans, W_start, b_start, W_gate, b_gate, W_noise, b_noise, train):
    return _run(*_prep(x_trans, W_start, b_start, W_gate, b_gate))


# speedup vs baseline: 5.8983x; 1.0008x over previous
import functools

import jax
import jax.numpy as jnp
from jax.experimental import pallas as pl

N_TOK, D_MODEL, N_NODES, N_EXPERTS = 8192, 1024, 16, 64
BLOCK_N = 256


def _router_body(x_ref, wsv_ref, bst_ref, wgt_ref, bg_ref, gates_ref, load_ref):
    xr = x_ref[...].astype(jnp.bfloat16).astype(jnp.float32)
    p = xr * wsv_ref[...]
    x1 = jnp.sum(p, axis=1) + bst_ref[0, 0]
    logits = (
        jnp.dot(x1.astype(jnp.bfloat16), wgt_ref[...],
                preferred_element_type=jnp.float32)
        + bg_ref[...]
    )
    col = jax.lax.broadcasted_iota(jnp.int32, logits.shape, 1)
    m1 = jnp.max(logits, axis=1, keepdims=True)
    i1 = jnp.min(jnp.where(logits == m1, col, N_EXPERTS), axis=1, keepdims=True)
    masked = jnp.where(col == i1, -jnp.inf, logits)
    m2 = jnp.max(masked, axis=1, keepdims=True)
    i2 = jnp.min(jnp.where(masked == m2, col, N_EXPERTS), axis=1, keepdims=True)
    t = jnp.exp(m2 - m1)
    denom = 1.0 + t
    g1 = 1.0 / denom
    g2 = t / denom
    gates = jnp.where(col == i1, g1, 0.0) + jnp.where(col == i2, g2, 0.0)
    gates_ref[...] = gates
    part = jnp.sum((gates > 0.0).astype(jnp.int32), axis=0, keepdims=True)

    @pl.when(pl.program_id(0) == 0)
    def _init():
        load_ref[...] = part

    @pl.when(pl.program_id(0) != 0)
    def _acc():
        load_ref[...] += part


@functools.partial(jax.jit, static_argnames=("interpret",))
def _run(xv, wsv, bst, wgt, bg, interpret=False):
    grid = (N_TOK // BLOCK_N,)
    gates, load = pl.pallas_call(
        _router_body,
        grid=grid,
        in_specs=[
            pl.BlockSpec((BLOCK_N, N_NODES, D_MODEL), lambda i: (i, 0, 0)),
            pl.BlockSpec((1, N_NODES, 1), lambda i: (0, 0, 0)),
            pl.BlockSpec((1, 1), lambda i: (0, 0)),
            pl.BlockSpec((D_MODEL, N_EXPERTS), lambda i: (0, 0)),
            pl.BlockSpec((1, N_EXPERTS), lambda i: (0, 0)),
        ],
        out_specs=[
            pl.BlockSpec((BLOCK_N, N_EXPERTS), lambda i: (i, 0)),
            pl.BlockSpec((1, N_EXPERTS), lambda i: (0, 0)),
        ],
        out_shape=[
            jax.ShapeDtypeStruct((N_TOK, N_EXPERTS), jnp.float32),
            jax.ShapeDtypeStruct((1, N_EXPERTS), jnp.int32),
        ],
        interpret=interpret,
    )(xv, wsv, bst, wgt, bg)
    return gates, load[0]


def _prep(x_trans, W_start, b_start, W_gate, b_gate):
    xv = jnp.transpose(x_trans, (0, 2, 1))
    wsv = jax.lax.reduce_precision(W_start[0], 8, 7).reshape(1, N_NODES, 1)
    bst = jnp.reshape(b_start[0], (1, 1)).astype(jnp.float32)
    wgt = W_gate.T.astype(jnp.bfloat16)
    bg = b_gate.astype(jnp.float32)[None, :]
    return xv, wsv, bst, wgt, bg


def kernel(x_trans, W_start, b_start, W_gate, b_gate, W_noise, b_noise, train):
    return _run(*_prep(x_trans, W_start, b_start, W_gate, b_gate))
